# baseline (speedup 1.0000x reference)
import jax
import jax.numpy as jnp
from jax.experimental import pallas as pl
from jax.experimental.pallas import tpu as pltpu

_BLK = 256


def _pe_add_kernel(x_ref, pe_ref, o_ref):
    o_ref[...] = x_ref[...] + pe_ref[...][None, :, :]


def _half(x, pe_rows, b0, nb, L, E):
    return pl.pallas_call(
        _pe_add_kernel,
        grid=(L // _BLK,),
        in_specs=[
            pl.BlockSpec((nb, _BLK, E), lambda j: (b0, j, 0)),
            pl.BlockSpec((_BLK, E), lambda j: (j, 0)),
        ],
        out_specs=pl.BlockSpec((nb, _BLK, E), lambda j: (0, j, 0)),
        out_shape=jax.ShapeDtypeStruct((nb, L, E), x.dtype),
    )(x, pe_rows)


def kernel(x, pe):
    B, L, E = x.shape
    pe_rows = jax.lax.slice(pe, (1, 0), (1 + L, E))
    lo = _half(x, pe_rows, 0, B // 2, L, E)
    hi = _half(x, pe_rows, 1, B // 2, L, E)
    return jnp.concatenate([lo, hi], axis=0)


# speedup vs baseline: 1.8570x; 1.8570x over previous
import jax
import jax.numpy as jnp
from jax.experimental import pallas as pl
from jax.experimental.pallas import tpu as pltpu

_BLK = 256


def _pe_add_kernel(x_ref, pe_ref, o_ref):
    o_ref[...] = x_ref[...] + pe_ref[...][None, :, :]


def kernel(x, pe):
    B, L, E = x.shape
    pe_rows = jax.lax.slice(pe, (1, 0), (1 + L, E))
    return pl.pallas_call(
        _pe_add_kernel,
        grid=(L // _BLK,),
        in_specs=[
            pl.BlockSpec((B, _BLK, E), lambda j: (0, j, 0)),
            pl.BlockSpec((_BLK, E), lambda j: (j, 0)),
        ],
        out_specs=pl.BlockSpec((B, _BLK, E), lambda j: (0, j, 0)),
        out_shape=jax.ShapeDtypeStruct((B, L, E), x.dtype),
        compiler_params=pltpu.CompilerParams(
            dimension_semantics=("parallel",),
        ),
    )(x, pe_rows)
